# baseline (speedup 1.0000x reference)
import jax
import jax.numpy as jnp
from jax import lax
from jax.experimental import pallas as pl
from jax.experimental.pallas import tpu as pltpu
from jax.experimental.pallas import tpu_sc as plsc

N, E, D, P = 10000, 160000, 1024, 256

NC, NS, L = 2, 16, 16
NW = NC * NS
RT = 88
NPASS = 4
NOUT = NPASS * NW * RT
GROW = RT + 8
ECAP = 4096
ETHRESH = 896
C = 3200
NCHUNK = E // C
XROWS = N + 16


def _sc_body(xp_hbm, esrc_hbm, edst_hbm, z_hbm, agg_hbm, deg_hbm,
             ec_s0, ec_d0, els, eld, idxb, degt, degf,
             buf, acc, sem_s0, sem_d0, sem_g):
    c = lax.axis_index("c")
    s = lax.axis_index("s")
    wid = s * NC + c
    lane = lax.iota(jnp.int32, L)
    zero16 = jnp.zeros((L,), jnp.int32)
    one_l0 = jnp.where(lane == 0, 1, 0).astype(jnp.int32)

    def _pf(i, carry):
        els[pl.ds(i * L, L)] = ((i * L + lane) * 61) % (N - 1)
        return carry
    lax.fori_loop(0, ECAP // L, _pf, 0)

    def scan_groups(src_ref, dst_ref, lo, cnt):
        def _g(j, cnt):
            d = dst_ref[pl.ds(j * L, L)]
            sv = src_ref[pl.ds(j * L, L)]
            ld = d - lo
            m = ld.astype(jnp.uint32) < jnp.uint32(RT)
            mi = m.astype(jnp.int32)
            posi = cnt + plsc.cumsum(mi) - 1
            plsc.store_scatter(els, [posi], sv, mask=m)
            plsc.store_scatter(eld, [posi], ld, mask=m)
            return cnt + jnp.sum(mi)
        return lax.fori_loop(0, C // L, _g, cnt)

    def flush(cnt, thresh):
        def w_cond(cv):
            return cv >= thresh

        def w_body(cv):
            def _g(g, carry):
                idxb[pl.ds(0, L)] = els[pl.ds(g * L, L)]
                pltpu.sync_copy(xp_hbm.at[idxb], buf)
                ldv = eld[pl.ds(g * L, L)]
                for j2 in range(L):
                    validj = (g * L + j2) < cv
                    ldj = jnp.sum(jnp.where(lane == j2, ldv, 0))
                    ld_eff = jnp.where(validj, ldj, RT)

                    def _add(k, carry2, j2=j2, ld_eff=ld_eff):
                        for u in range(16):
                            off = k * 256 + u * L
                            acc[ld_eff, pl.ds(off, L)] += buf[j2, pl.ds(off, L)]
                        return carry2

                    lax.fori_loop(0, D // 256, _add, 0)
                    plsc.addupdate_scatter(degt, [lane * 0 + ld_eff], one_l0,
                                           mask=lane == 0)
                return carry
            lax.fori_loop(0, (cv + L - 1) // L, _g, 0)
            return jnp.int32(0)

        return lax.while_loop(w_cond, w_body, cnt)

    def _pass(p, carry0):
        lo = pl.multiple_of((p * NW + wid) * RT, 8)
        pltpu.sync_copy(z_hbm, acc.at[pl.ds(0, RT)])

        def _zd(i, carry):
            degt[pl.ds(i * L, L)] = zero16
            return carry
        lax.fori_loop(0, GROW // L, _zd, 0)

        def _step(k, cnt):
            b = k * C
            pltpu.async_copy(esrc_hbm.at[pl.ds(b, C)], ec_s0, sem_s0).wait()
            pltpu.async_copy(edst_hbm.at[pl.ds(b, C)], ec_d0, sem_d0).wait()
            cnt = scan_groups(ec_s0, ec_d0, lo, cnt)
            return flush(cnt, ETHRESH)

        cnt = lax.fori_loop(0, NCHUNK, _step, jnp.int32(0))
        flush(cnt, 1)

        pltpu.sync_copy(acc.at[pl.ds(0, RT)], agg_hbm.at[pl.ds(lo, RT)])

        def _cv(i, carry):
            degf[pl.ds(i * L, L)] = degt[pl.ds(i * L, L)].astype(jnp.float32)
            return carry
        lax.fori_loop(0, GROW // L, _cv, 0)
        pltpu.sync_copy(degf.at[pl.ds(0, RT)], deg_hbm.at[pl.ds(lo, RT)])
        return carry0

    lax.fori_loop(0, NPASS, _pass, 0)


def _sc_agg(xp, esrc, edst):
    f = pl.kernel(
        _sc_body,
        out_type=(jax.ShapeDtypeStruct((NOUT, D), jnp.float32),
                  jax.ShapeDtypeStruct((NOUT,), jnp.float32)),
        mesh=plsc.VectorSubcoreMesh(core_axis_name="c", subcore_axis_name="s",
                                    num_cores=NC, num_subcores=NS),
        scratch_types=[
            pltpu.VMEM((C,), jnp.int32),
            pltpu.VMEM((C,), jnp.int32),
            pltpu.VMEM((ECAP,), jnp.int32),
            pltpu.VMEM((ECAP,), jnp.int32),
            pltpu.VMEM((L,), jnp.int32),
            pltpu.VMEM((GROW,), jnp.int32),
            pltpu.VMEM((GROW,), jnp.float32),
            pltpu.VMEM((L, D), jnp.float32),
            pltpu.VMEM((GROW, D), jnp.float32),
            pltpu.SemaphoreType.DMA,
            pltpu.SemaphoreType.DMA,
            pltpu.SemaphoreType.DMA,
        ],
        compiler_params=pltpu.CompilerParams(needs_layout_passes=False),
    )
    z = jnp.zeros((RT, D), jnp.float32)
    return f(xp, esrc, edst, z)


BM = 1000
GRID = N // BM


def _mlp1_body(x_ref, agg_ref, deg_ref, wmp_ref, bmp_ref, w1_ref, b1_ref,
               y1_ref, s1_ref, q1_ref):
    i = pl.program_id(0)
    degc = jnp.maximum(deg_ref[...], 1.0)
    a = agg_ref[...] / degc
    h = x_ref[...] + 0.1 * (
        jnp.dot(a, wmp_ref[...], preferred_element_type=jnp.float32) + bmp_ref[...])
    y = jnp.dot(h, w1_ref[...], preferred_element_type=jnp.float32) + b1_ref[...]
    y1_ref[...] = y
    sy = jnp.sum(y, axis=0, keepdims=True)
    qy = jnp.sum(y * y, axis=0, keepdims=True)

    @pl.when(i == 0)
    def _():
        s1_ref[...] = sy
        q1_ref[...] = qy

    @pl.when(i != 0)
    def _():
        s1_ref[...] += sy
        q1_ref[...] += qy


def _mlp1(x, agg, deg, wmp, bmp, w1, b1):
    return pl.pallas_call(
        _mlp1_body,
        grid=(GRID,),
        in_specs=[
            pl.BlockSpec((BM, D), lambda i: (i, 0)),
            pl.BlockSpec((BM, D), lambda i: (i, 0)),
            pl.BlockSpec((BM, 1), lambda i: (i, 0)),
            pl.BlockSpec((D, D), lambda i: (0, 0)),
            pl.BlockSpec((1, D), lambda i: (0, 0)),
            pl.BlockSpec((D, D), lambda i: (0, 0)),
            pl.BlockSpec((1, D), lambda i: (0, 0)),
        ],
        out_specs=[
            pl.BlockSpec((BM, D), lambda i: (i, 0)),
            pl.BlockSpec((1, D), lambda i: (0, 0)),
            pl.BlockSpec((1, D), lambda i: (0, 0)),
        ],
        out_shape=[
            jax.ShapeDtypeStruct((N, D), jnp.float32),
            jax.ShapeDtypeStruct((1, D), jnp.float32),
            jax.ShapeDtypeStruct((1, D), jnp.float32),
        ],
    )(x, agg, deg, wmp, bmp, w1, b1)


def _mlp2_body(y1_ref, s1_ref, q1_ref, g1_ref, t1_ref, w2_ref, b2_ref,
               y2_ref, s2_ref, q2_ref):
    i = pl.program_id(0)
    mu = s1_ref[...] / N
    var = q1_ref[...] / N - mu * mu
    inv = g1_ref[...] * lax.rsqrt(var + 1e-5)
    a1 = jnp.maximum(inv * (y1_ref[...] - mu) + t1_ref[...], 0.0)
    y = jnp.dot(a1, w2_ref[...], preferred_element_type=jnp.float32) + b2_ref[...]
    y2_ref[...] = y
    sy = jnp.sum(y, axis=0, keepdims=True)
    qy = jnp.sum(y * y, axis=0, keepdims=True)

    @pl.when(i == 0)
    def _():
        s2_ref[...] = sy
        q2_ref[...] = qy

    @pl.when(i != 0)
    def _():
        s2_ref[...] += sy
        q2_ref[...] += qy


def _mlp2(y1, s1, q1, g1, t1, w2, b2):
    return pl.pallas_call(
        _mlp2_body,
        grid=(GRID,),
        in_specs=[
            pl.BlockSpec((BM, D), lambda i: (i, 0)),
            pl.BlockSpec((1, D), lambda i: (0, 0)),
            pl.BlockSpec((1, D), lambda i: (0, 0)),
            pl.BlockSpec((1, D), lambda i: (0, 0)),
            pl.BlockSpec((1, D), lambda i: (0, 0)),
            pl.BlockSpec((D, P), lambda i: (0, 0)),
            pl.BlockSpec((1, P), lambda i: (0, 0)),
        ],
        out_specs=[
            pl.BlockSpec((BM, P), lambda i: (i, 0)),
            pl.BlockSpec((1, P), lambda i: (0, 0)),
            pl.BlockSpec((1, P), lambda i: (0, 0)),
        ],
        out_shape=[
            jax.ShapeDtypeStruct((N, P), jnp.float32),
            jax.ShapeDtypeStruct((1, P), jnp.float32),
            jax.ShapeDtypeStruct((1, P), jnp.float32),
        ],
    )(y1, s1, q1, g1, t1, w2, b2)


def _bn2_body(y2_ref, s2_ref, q2_ref, g2_ref, t2_ref, emb_ref, nemb_ref):
    mu = s2_ref[...] / N
    var = q2_ref[...] / N - mu * mu
    inv = g2_ref[...] * lax.rsqrt(var + 1e-5)
    e = jnp.maximum(inv * (y2_ref[...] - mu) + t2_ref[...], 0.0)
    emb_ref[...] = e
    nrm = jnp.maximum(jnp.sqrt(jnp.sum(e * e, axis=1, keepdims=True)), 1e-12)
    nemb_ref[...] = e / nrm


def _bn2(y2, s2, q2, g2, t2):
    return pl.pallas_call(
        _bn2_body,
        grid=(GRID,),
        in_specs=[
            pl.BlockSpec((BM, P), lambda i: (i, 0)),
            pl.BlockSpec((1, P), lambda i: (0, 0)),
            pl.BlockSpec((1, P), lambda i: (0, 0)),
            pl.BlockSpec((1, P), lambda i: (0, 0)),
            pl.BlockSpec((1, P), lambda i: (0, 0)),
        ],
        out_specs=[
            pl.BlockSpec((BM, P), lambda i: (i, 0)),
            pl.BlockSpec((BM, P), lambda i: (i, 0)),
        ],
        out_shape=[
            jax.ShapeDtypeStruct((N, P), jnp.float32),
            jax.ShapeDtypeStruct((N, P), jnp.float32),
        ],
    )(y2, s2, q2, g2, t2)


BMA = 400


def _aff_body(a_ref, b_ref, o_ref):
    o_ref[...] = lax.dot_general(a_ref[...], b_ref[...], (((1,), (1,)), ((), ())),
                                 preferred_element_type=jnp.float32)


def _aff(nemb_s, nemb_t):
    return pl.pallas_call(
        _aff_body,
        grid=(N // BMA,),
        in_specs=[
            pl.BlockSpec((BMA, P), lambda i: (i, 0)),
            pl.BlockSpec((N, P), lambda i: (0, 0)),
        ],
        out_specs=pl.BlockSpec((BMA, N), lambda i: (i, 0)),
        out_shape=jax.ShapeDtypeStruct((N, N), jnp.float32),
    )(nemb_s, nemb_t)


def _proj_tc(x, agg, deg, W_mp, b_mp, W1, b1, g1, t1, W2, b2, g2, t2):
    y1, s1, q1 = _mlp1(x, agg, deg, W_mp, b_mp.reshape(1, D), W1, b1.reshape(1, D))
    y2, s2, q2 = _mlp2(y1, s1, q1, g1.reshape(1, D), t1.reshape(1, D),
                       W2, b2.reshape(1, P))
    return _bn2(y2, s2, q2, g2.reshape(1, P), t2.reshape(1, P))


def kernel(x_src, edge_index_src, x_tgt, edge_index_tgt,
           W_mp, b_mp, W1, b1, g1, t1, W2, b2, g2, t2):
    zpad = jnp.zeros((XROWS - N, D), jnp.float32)
    xps = jnp.concatenate([x_src, zpad], axis=0)
    xpt = jnp.concatenate([x_tgt, zpad], axis=0)
    agg_s, deg_s = _sc_agg(xps, edge_index_src[0], edge_index_src[1])
    agg_t, deg_t = _sc_agg(xpt, edge_index_tgt[0], edge_index_tgt[1])
    emb_s, nemb_s = _proj_tc(x_src, agg_s, deg_s.reshape(NOUT, 1),
                             W_mp, b_mp, W1, b1, g1, t1, W2, b2, g2, t2)
    emb_t, nemb_t = _proj_tc(x_tgt, agg_t, deg_t.reshape(NOUT, 1),
                             W_mp, b_mp, W1, b1, g1, t1, W2, b2, g2, t2)
    affinity = _aff(nemb_s, nemb_t)
    return (emb_s, emb_t, affinity)


# speedup vs baseline: 1.0569x; 1.0569x over previous
import jax
import jax.numpy as jnp
from jax import lax
from jax.experimental import pallas as pl
from jax.experimental.pallas import tpu as pltpu
from jax.experimental.pallas import tpu_sc as plsc

N, E, D, P = 10000, 160000, 1024, 256

NC, NS, L = 2, 16, 16
NW = NC * NS
RT = 88
NPASS = 4
NOUT = NPASS * NW * RT
GROW = RT + 8
ECAP = 3072
ETHRESH = 1024
C = 2000
NCHUNK = E // C
XROWS = N + 16


def _sc_body(xp_hbm, esrc_hbm, edst_hbm, z_hbm, agg_hbm, deg_hbm,
             ec_s0, ec_d0, els, eld, idxb, degt, degf,
             buf, acc, sem_s0, sem_d0, sem_g):
    c = lax.axis_index("c")
    s = lax.axis_index("s")
    wid = s * NC + c
    lane = lax.iota(jnp.int32, L)
    zero16 = jnp.zeros((L,), jnp.int32)
    one_l0 = jnp.where(lane == 0, 1, 0).astype(jnp.int32)

    def _pf(i, carry):
        els[pl.ds(i * L, L)] = ((i * L + lane) * 61) % (N - 1)
        return carry
    lax.fori_loop(0, ECAP // L, _pf, 0)

    def scan_groups(src_ref, dst_ref, lo, cnt):
        def _g(j, cnt):
            d = dst_ref[pl.ds(j * L, L)]
            sv = src_ref[pl.ds(j * L, L)]
            ld = d - lo
            m = ld.astype(jnp.uint32) < jnp.uint32(RT)
            mi = m.astype(jnp.int32)
            posi = cnt + plsc.cumsum(mi) - 1
            plsc.store_scatter(els, [posi], sv, mask=m)
            plsc.store_scatter(eld, [posi], ld, mask=m)
            return cnt + jnp.sum(mi)
        return lax.fori_loop(0, C // L, _g, cnt)

    def flush(cnt, thresh):
        def w_cond(cv):
            return cv >= thresh

        def w_body(cv):
            def _g(g, carry):
                idxb[pl.ds(0, L)] = els[pl.ds(g * L, L)]
                pltpu.sync_copy(xp_hbm.at[idxb], buf)
                ldv = eld[pl.ds(g * L, L)]
                for j2 in range(L):
                    validj = (g * L + j2) < cv
                    ldj = jnp.sum(jnp.where(lane == j2, ldv, 0))
                    ld_eff = jnp.where(validj, ldj, RT)

                    def _add(k, carry2, j2=j2, ld_eff=ld_eff):
                        for u in range(8):
                            off = k * 128 + u * L
                            acc[ld_eff, pl.ds(off, L)] += buf[j2, pl.ds(off, L)]
                        return carry2

                    lax.fori_loop(0, D // 128, _add, 0)
                    plsc.addupdate_scatter(degt, [lane * 0 + ld_eff], one_l0,
                                           mask=lane == 0)
                return carry
            lax.fori_loop(0, (cv + L - 1) // L, _g, 0)
            return jnp.int32(0)

        return lax.while_loop(w_cond, w_body, cnt)

    def _pass(p, carry0):
        lo = pl.multiple_of((p * NW + wid) * RT, 8)
        pltpu.sync_copy(z_hbm, acc.at[pl.ds(0, RT)])

        def _zd(i, carry):
            degt[pl.ds(i * L, L)] = zero16
            return carry
        lax.fori_loop(0, GROW // L, _zd, 0)

        def _step(k, cnt):
            b = k * C
            pltpu.async_copy(esrc_hbm.at[pl.ds(b, C)], ec_s0, sem_s0).wait()
            pltpu.async_copy(edst_hbm.at[pl.ds(b, C)], ec_d0, sem_d0).wait()
            cnt = scan_groups(ec_s0, ec_d0, lo, cnt)
            return flush(cnt, ETHRESH)

        cnt = lax.fori_loop(0, NCHUNK, _step, jnp.int32(0))
        flush(cnt, 1)

        pltpu.sync_copy(acc.at[pl.ds(0, RT)], agg_hbm.at[pl.ds(lo, RT)])

        def _cv(i, carry):
            degf[pl.ds(i * L, L)] = degt[pl.ds(i * L, L)].astype(jnp.float32)
            return carry
        lax.fori_loop(0, GROW // L, _cv, 0)
        pltpu.sync_copy(degf.at[pl.ds(0, RT)], deg_hbm.at[pl.ds(lo, RT)])
        return carry0

    lax.fori_loop(0, NPASS, _pass, 0)


def _sc_agg(xp, esrc, edst):
    f = pl.kernel(
        _sc_body,
        out_type=(jax.ShapeDtypeStruct((NOUT, D), jnp.float32),
                  jax.ShapeDtypeStruct((NOUT,), jnp.float32)),
        mesh=plsc.VectorSubcoreMesh(core_axis_name="c", subcore_axis_name="s",
                                    num_cores=NC, num_subcores=NS),
        scratch_types=[
            pltpu.VMEM((C,), jnp.int32),
            pltpu.VMEM((C,), jnp.int32),
            pltpu.VMEM((ECAP,), jnp.int32),
            pltpu.VMEM((ECAP,), jnp.int32),
            pltpu.VMEM((L,), jnp.int32),
            pltpu.VMEM((GROW,), jnp.int32),
            pltpu.VMEM((GROW,), jnp.float32),
            pltpu.VMEM((L, D), jnp.float32),
            pltpu.VMEM((GROW, D), jnp.float32),
            pltpu.SemaphoreType.DMA,
            pltpu.SemaphoreType.DMA,
            pltpu.SemaphoreType.DMA,
        ],
        compiler_params=pltpu.CompilerParams(needs_layout_passes=False),
    )
    z = jnp.zeros((RT, D), jnp.float32)
    return f(xp, esrc, edst, z)


BM = 1000
GRID = N // BM


def _mlp1_body(x_ref, agg_ref, deg_ref, wmp_ref, bmp_ref, w1_ref, b1_ref,
               y1_ref, s1_ref, q1_ref):
    i = pl.program_id(0)
    degc = jnp.maximum(deg_ref[...], 1.0)
    a = agg_ref[...] / degc
    h = x_ref[...] + 0.1 * (
        jnp.dot(a, wmp_ref[...], preferred_element_type=jnp.float32) + bmp_ref[...])
    y = jnp.dot(h, w1_ref[...], preferred_element_type=jnp.float32) + b1_ref[...]
    y1_ref[...] = y
    sy = jnp.sum(y, axis=0, keepdims=True)
    qy = jnp.sum(y * y, axis=0, keepdims=True)

    @pl.when(i == 0)
    def _():
        s1_ref[...] = sy
        q1_ref[...] = qy

    @pl.when(i != 0)
    def _():
        s1_ref[...] += sy
        q1_ref[...] += qy


def _mlp1(x, agg, deg, wmp, bmp, w1, b1):
    return pl.pallas_call(
        _mlp1_body,
        grid=(GRID,),
        in_specs=[
            pl.BlockSpec((BM, D), lambda i: (i, 0)),
            pl.BlockSpec((BM, D), lambda i: (i, 0)),
            pl.BlockSpec((BM, 1), lambda i: (i, 0)),
            pl.BlockSpec((D, D), lambda i: (0, 0)),
            pl.BlockSpec((1, D), lambda i: (0, 0)),
            pl.BlockSpec((D, D), lambda i: (0, 0)),
            pl.BlockSpec((1, D), lambda i: (0, 0)),
        ],
        out_specs=[
            pl.BlockSpec((BM, D), lambda i: (i, 0)),
            pl.BlockSpec((1, D), lambda i: (0, 0)),
            pl.BlockSpec((1, D), lambda i: (0, 0)),
        ],
        out_shape=[
            jax.ShapeDtypeStruct((N, D), jnp.float32),
            jax.ShapeDtypeStruct((1, D), jnp.float32),
            jax.ShapeDtypeStruct((1, D), jnp.float32),
        ],
    )(x, agg, deg, wmp, bmp, w1, b1)


def _mlp2_body(y1_ref, s1_ref, q1_ref, g1_ref, t1_ref, w2_ref, b2_ref,
               y2_ref, s2_ref, q2_ref):
    i = pl.program_id(0)
    mu = s1_ref[...] / N
    var = q1_ref[...] / N - mu * mu
    inv = g1_ref[...] * lax.rsqrt(var + 1e-5)
    a1 = jnp.maximum(inv * (y1_ref[...] - mu) + t1_ref[...], 0.0)
    y = jnp.dot(a1, w2_ref[...], preferred_element_type=jnp.float32) + b2_ref[...]
    y2_ref[...] = y
    sy = jnp.sum(y, axis=0, keepdims=True)
    qy = jnp.sum(y * y, axis=0, keepdims=True)

    @pl.when(i == 0)
    def _():
        s2_ref[...] = sy
        q2_ref[...] = qy

    @pl.when(i != 0)
    def _():
        s2_ref[...] += sy
        q2_ref[...] += qy


def _mlp2(y1, s1, q1, g1, t1, w2, b2):
    return pl.pallas_call(
        _mlp2_body,
        grid=(GRID,),
        in_specs=[
            pl.BlockSpec((BM, D), lambda i: (i, 0)),
            pl.BlockSpec((1, D), lambda i: (0, 0)),
            pl.BlockSpec((1, D), lambda i: (0, 0)),
            pl.BlockSpec((1, D), lambda i: (0, 0)),
            pl.BlockSpec((1, D), lambda i: (0, 0)),
            pl.BlockSpec((D, P), lambda i: (0, 0)),
            pl.BlockSpec((1, P), lambda i: (0, 0)),
        ],
        out_specs=[
            pl.BlockSpec((BM, P), lambda i: (i, 0)),
            pl.BlockSpec((1, P), lambda i: (0, 0)),
            pl.BlockSpec((1, P), lambda i: (0, 0)),
        ],
        out_shape=[
            jax.ShapeDtypeStruct((N, P), jnp.float32),
            jax.ShapeDtypeStruct((1, P), jnp.float32),
            jax.ShapeDtypeStruct((1, P), jnp.float32),
        ],
    )(y1, s1, q1, g1, t1, w2, b2)


def _bn2_body(y2_ref, s2_ref, q2_ref, g2_ref, t2_ref, emb_ref, nemb_ref):
    mu = s2_ref[...] / N
    var = q2_ref[...] / N - mu * mu
    inv = g2_ref[...] * lax.rsqrt(var + 1e-5)
    e = jnp.maximum(inv * (y2_ref[...] - mu) + t2_ref[...], 0.0)
    emb_ref[...] = e
    nrm = jnp.maximum(jnp.sqrt(jnp.sum(e * e, axis=1, keepdims=True)), 1e-12)
    nemb_ref[...] = e / nrm


def _bn2(y2, s2, q2, g2, t2):
    return pl.pallas_call(
        _bn2_body,
        grid=(GRID,),
        in_specs=[
            pl.BlockSpec((BM, P), lambda i: (i, 0)),
            pl.BlockSpec((1, P), lambda i: (0, 0)),
            pl.BlockSpec((1, P), lambda i: (0, 0)),
            pl.BlockSpec((1, P), lambda i: (0, 0)),
            pl.BlockSpec((1, P), lambda i: (0, 0)),
        ],
        out_specs=[
            pl.BlockSpec((BM, P), lambda i: (i, 0)),
            pl.BlockSpec((BM, P), lambda i: (i, 0)),
        ],
        out_shape=[
            jax.ShapeDtypeStruct((N, P), jnp.float32),
            jax.ShapeDtypeStruct((N, P), jnp.float32),
        ],
    )(y2, s2, q2, g2, t2)


BMA = 400


def _aff_body(a_ref, b_ref, o_ref):
    o_ref[...] = lax.dot_general(a_ref[...], b_ref[...], (((1,), (1,)), ((), ())),
                                 preferred_element_type=jnp.float32)


def _aff(nemb_s, nemb_t):
    return pl.pallas_call(
        _aff_body,
        grid=(N // BMA,),
        in_specs=[
            pl.BlockSpec((BMA, P), lambda i: (i, 0)),
            pl.BlockSpec((N, P), lambda i: (0, 0)),
        ],
        out_specs=pl.BlockSpec((BMA, N), lambda i: (i, 0)),
        out_shape=jax.ShapeDtypeStruct((N, N), jnp.float32),
    )(nemb_s, nemb_t)


def _proj_tc(x, agg, deg, W_mp, b_mp, W1, b1, g1, t1, W2, b2, g2, t2):
    y1, s1, q1 = _mlp1(x, agg, deg, W_mp, b_mp.reshape(1, D), W1, b1.reshape(1, D))
    y2, s2, q2 = _mlp2(y1, s1, q1, g1.reshape(1, D), t1.reshape(1, D),
                       W2, b2.reshape(1, P))
    return _bn2(y2, s2, q2, g2.reshape(1, P), t2.reshape(1, P))


def kernel(x_src, edge_index_src, x_tgt, edge_index_tgt,
           W_mp, b_mp, W1, b1, g1, t1, W2, b2, g2, t2):
    zpad = jnp.zeros((XROWS - N, D), jnp.float32)
    xps = jnp.concatenate([x_src, zpad], axis=0)
    xpt = jnp.concatenate([x_tgt, zpad], axis=0)
    agg_s, deg_s = _sc_agg(xps, edge_index_src[0], edge_index_src[1])
    agg_t, deg_t = _sc_agg(xpt, edge_index_tgt[0], edge_index_tgt[1])
    emb_s, nemb_s = _proj_tc(x_src, agg_s, deg_s.reshape(NOUT, 1),
                             W_mp, b_mp, W1, b1, g1, t1, W2, b2, g2, t2)
    emb_t, nemb_t = _proj_tc(x_tgt, agg_t, deg_t.reshape(NOUT, 1),
                             W_mp, b_mp, W1, b1, g1, t1, W2, b2, g2, t2)
    affinity = _aff(nemb_s, nemb_t)
    return (emb_s, emb_t, affinity)
